# baseline (speedup 1.0000x reference)
import functools

import jax
import jax.numpy as jnp
from jax import lax
from jax.experimental import pallas as pl
from jax.experimental.pallas import tpu as pltpu
from jax.experimental.pallas import tpu_sc as plsc

_BATCH = 16384
_EMB = 32
_GRP = 128
_NC = 2
_NS = 16
_NW = _NC * _NS
_BPW = _BATCH // _NW
_CHUNK = 256

_mesh = plsc.VectorSubcoreMesh(core_axis_name="c", subcore_axis_name="s")


@functools.partial(
    pl.kernel,
    mesh=_mesh,
    out_type=[
        jax.ShapeDtypeStruct((_BATCH, _GRP), jnp.float32),
        jax.ShapeDtypeStruct((_BATCH, _GRP), jnp.float32),
    ],
    scratch_types=[
        pltpu.VMEM((_BPW,), jnp.int32),
        pltpu.VMEM((_BPW,), jnp.int32),
        pltpu.VMEM((_CHUNK, _GRP), jnp.float32),
        pltpu.VMEM((_CHUNK, _GRP), jnp.float32),
        pltpu.SemaphoreType.DMA,
        pltpu.SemaphoreType.DMA,
    ],
)
def _sc_gather(uid_hbm, mid_hbm, uemb_hbm, memb_hbm, gu_hbm, gm_hbm,
               uidx, midx, ubuf, mbuf, sem_u, sem_m):
    wid = lax.axis_index("s") * _NC + lax.axis_index("c")
    base = wid * _BPW
    pltpu.sync_copy(uid_hbm.at[pl.ds(base, _BPW)], uidx)
    pltpu.sync_copy(mid_hbm.at[pl.ds(base, _BPW)], midx)
    for j in range(_BPW // 16):
        sl = pl.ds(16 * j, 16)
        uidx[sl] = lax.shift_right_logical(uidx[sl], 2)
        midx[sl] = lax.shift_right_logical(midx[sl], 2)
    for c in range(_BPW // _CHUNK):
        off = c * _CHUNK
        cu = pltpu.async_copy(uemb_hbm.at[uidx.at[pl.ds(off, _CHUNK)]], ubuf, sem_u)
        cm = pltpu.async_copy(memb_hbm.at[midx.at[pl.ds(off, _CHUNK)]], mbuf, sem_m)
        cu.wait()
        cm.wait()
        pltpu.sync_copy(ubuf, gu_hbm.at[pl.ds(base + off, _CHUNK)])
        pltpu.sync_copy(mbuf, gm_hbm.at[pl.ds(base + off, _CHUNK)])


_BLK = 2048


def _mlp_body(uid, mid, gu, gm, w1u, w1m, b1, w2, b2, w3, b3, w4, b4, out):
    su = uid[...] & 3
    sm = mid[...] & 3
    uf = jnp.zeros((_BLK, _EMB), jnp.float32)
    mf = jnp.zeros((_BLK, _EMB), jnp.float32)
    for k in range(4):
        uf = uf + jnp.where(su == k, gu[:, 32 * k:32 * k + 32], 0.0)
        mf = mf + jnp.where(sm == k, gm[:, 32 * k:32 * k + 32], 0.0)
    x = jnp.dot(uf, w1u[...], preferred_element_type=jnp.float32)
    x = x + jnp.dot(mf, w1m[...], preferred_element_type=jnp.float32)
    x = jnp.maximum(x + b1[...][None, :], 0.0)
    x = jnp.maximum(
        jnp.dot(x, w2[...], preferred_element_type=jnp.float32) + b2[...][None, :], 0.0)
    x = jnp.maximum(
        jnp.dot(x, w3[...], preferred_element_type=jnp.float32) + b3[...][None, :], 0.0)
    y = jnp.dot(x, w4[...], preferred_element_type=jnp.float32)
    out[...] = y[:, 0] + b4[...]


_mlp_call = pl.pallas_call(
    _mlp_body,
    grid=(_BATCH // _BLK,),
    in_specs=[
        pl.BlockSpec((_BLK, 1), lambda i: (i, 0)),
        pl.BlockSpec((_BLK, 1), lambda i: (i, 0)),
        pl.BlockSpec((_BLK, _GRP), lambda i: (i, 0)),
        pl.BlockSpec((_BLK, _GRP), lambda i: (i, 0)),
        pl.BlockSpec((_EMB, 32), lambda i: (0, 0)),
        pl.BlockSpec((_EMB, 32), lambda i: (0, 0)),
        pl.BlockSpec((32,), lambda i: (0,)),
        pl.BlockSpec((32, 16), lambda i: (0, 0)),
        pl.BlockSpec((16,), lambda i: (0,)),
        pl.BlockSpec((16, 8), lambda i: (0, 0)),
        pl.BlockSpec((8,), lambda i: (0,)),
        pl.BlockSpec((8, 1), lambda i: (0, 0)),
        pl.BlockSpec((1,), lambda i: (0,)),
    ],
    out_specs=pl.BlockSpec((_BLK,), lambda i: (i,)),
    out_shape=jax.ShapeDtypeStruct((_BATCH,), jnp.float32),
)


def kernel(user_id, movie_id, user_emb, movie_emb, W1, b1, W2, b2, W3, b3, W4, b4):
    uid = user_id.astype(jnp.int32)
    mid = movie_id.astype(jnp.int32)
    gu, gm = _sc_gather(uid, mid, user_emb.reshape(-1, _GRP),
                        movie_emb.reshape(-1, _GRP))
    return _mlp_call(uid.reshape(-1, 1), mid.reshape(-1, 1), gu, gm,
                     W1[:_EMB], W1[_EMB:], b1, W2, b2, W3, b3, W4, b4)


# speedup vs baseline: 2.3739x; 2.3739x over previous
import functools

import jax
import jax.numpy as jnp
from jax import lax
from jax.experimental import pallas as pl
from jax.experimental.pallas import tpu as pltpu
from jax.experimental.pallas import tpu_sc as plsc

_BATCH = 16384
_EMB = 32
_LANE = 128
_USERS = 1000000
_MOVIES = 100000
_NC = 2
_NS = 16
_NW = _NC * _NS
_BPW = _BATCH // _NW
_G = 4
_NGRP = _BPW // _G
_SROWS = 128

_UCMAX = ((_USERS - 1) // _LANE) * _LANE
_MCMAX = ((_MOVIES - 1) // _LANE) * _LANE

_mesh = plsc.VectorSubcoreMesh(core_axis_name="c", subcore_axis_name="s")


@functools.partial(
    pl.kernel,
    mesh=_mesh,
    compiler_params=pltpu.CompilerParams(needs_layout_passes=False),
    out_type=jax.ShapeDtypeStruct((_BATCH, _LANE), jnp.float32),
    scratch_types=[
        pltpu.VMEM((_BPW + 16,), jnp.int32),
        pltpu.VMEM((_BPW + 16,), jnp.int32),
        pltpu.VMEM((2, _G, _EMB, _LANE), jnp.float32),
        pltpu.VMEM((2, _G, _EMB, _LANE), jnp.float32),
        pltpu.VMEM((_SROWS, _LANE), jnp.float32),
        pltpu.SemaphoreType.DMA,
        pltpu.SemaphoreType.DMA,
        pltpu.SemaphoreType.DMA,
        pltpu.SemaphoreType.DMA,
    ],
)
def _sc_gather(uid_hbm, mid_hbm, ut_hbm, mt_hbm, out_hbm,
               uidx, midx, ublk, mblk, stage, su0, su1, sm0, sm1):
    wid = lax.axis_index("s") * _NC + lax.axis_index("c")
    base = wid * _BPW
    pltpu.sync_copy(uid_hbm.at[pl.ds(base, _BPW)], uidx.at[pl.ds(0, _BPW)])
    pltpu.sync_copy(mid_hbm.at[pl.ds(base, _BPW)], midx.at[pl.ds(0, _BPW)])
    sems_u = (su0, su1)
    sems_m = (sm0, sm1)
    iota = lax.iota(jnp.int32, 16)

    def blk_start(v, cmax):
        return jnp.minimum(v - (v & (_LANE - 1)), cmax)

    def fire(g, slot):
        uv = uidx[pl.ds(g * _G, 16)]
        mv = midx[pl.ds(g * _G, 16)]
        for j in range(_G):
            u = uv[j]
            cs = pl.multiple_of(blk_start(u, _UCMAX), _LANE)
            pltpu.async_copy(ut_hbm.at[:, pl.ds(cs, _LANE)],
                             ublk.at[slot, j], sems_u[slot])
            m = mv[j]
            ms = pl.multiple_of(blk_start(m, _MCMAX), _LANE)
            pltpu.async_copy(mt_hbm.at[:, pl.ds(ms, _LANE)],
                             mblk.at[slot, j], sems_m[slot])

    def drain(slot):
        for j in range(_G):
            pltpu.make_async_copy(ut_hbm.at[:, pl.ds(0, _LANE)],
                                  ublk.at[slot, j], sems_u[slot]).wait()
            pltpu.make_async_copy(mt_hbm.at[:, pl.ds(0, _LANE)],
                                  mblk.at[slot, j], sems_m[slot]).wait()

    def extract(blk_ref, col, row, colbase):
        cvec = jnp.full((16,), col, dtype=jnp.int32)
        rvec = jnp.full((16,), row, dtype=jnp.int32)
        lo = plsc.load_gather(blk_ref, [iota, cvec])
        hi = plsc.load_gather(blk_ref, [iota + 16, cvec])
        plsc.store_scatter(stage, [rvec, iota + colbase], lo)
        plsc.store_scatter(stage, [rvec, iota + (colbase + 16)], hi)

    def process(g, slot):
        uv = uidx[pl.ds(g * _G, 16)]
        mv = midx[pl.ds(g * _G, 16)]
        for j in range(_G):
            p = g * _G + j
            row = p & (_SROWS - 1)
            u = uv[j]
            ucol = u - blk_start(u, _UCMAX)
            extract(ublk.at[slot, j], ucol, row, 0)
            m = mv[j]
            mcol = m - blk_start(m, _MCMAX)
            extract(mblk.at[slot, j], mcol, row, 32)

    fire(0, 0)

    def body(i, carry):
        fire(2 * i + 1, 1)
        drain(0)
        process(2 * i, 0)

        @pl.when(2 * i + 2 < _NGRP)
        def _():
            fire(2 * i + 2, 0)

        drain(1)
        process(2 * i + 1, 1)

        @pl.when((i & 15) == 15)
        def _():
            off = pl.multiple_of(base + (i // 16) * _SROWS, _SROWS)
            pltpu.sync_copy(stage, out_hbm.at[pl.ds(off, _SROWS)])

        return carry

    lax.fori_loop(0, _NGRP // 2, body, 0)


_BLK = 2048


def _mlp_body(gath, w1, b1, w2, b2, w3, b3, w4, b4, out):
    uf = gath[:, 0:_EMB]
    mf = gath[:, _EMB:2 * _EMB]
    w1u = w1[0:_EMB, :]
    w1m = w1[_EMB:2 * _EMB, :]
    x = jnp.dot(uf, w1u, preferred_element_type=jnp.float32)
    x = x + jnp.dot(mf, w1m, preferred_element_type=jnp.float32)
    x = jnp.maximum(x + b1[...][None, :], 0.0)
    x = jnp.maximum(
        jnp.dot(x, w2[...], preferred_element_type=jnp.float32)
        + b2[...][None, :], 0.0)
    x = jnp.maximum(
        jnp.dot(x, w3[...], preferred_element_type=jnp.float32)
        + b3[...][None, :], 0.0)
    y = jnp.dot(x, w4[...], preferred_element_type=jnp.float32)
    out[...] = y[:, 0] + b4[...]


_mlp_call = pl.pallas_call(
    _mlp_body,
    grid=(_BATCH // _BLK,),
    in_specs=[
        pl.BlockSpec((_BLK, _LANE), lambda i: (i, 0)),
        pl.BlockSpec((2 * _EMB, 32), lambda i: (0, 0)),
        pl.BlockSpec((32,), lambda i: (0,)),
        pl.BlockSpec((32, 16), lambda i: (0, 0)),
        pl.BlockSpec((16,), lambda i: (0,)),
        pl.BlockSpec((16, 8), lambda i: (0, 0)),
        pl.BlockSpec((8,), lambda i: (0,)),
        pl.BlockSpec((8, 1), lambda i: (0, 0)),
        pl.BlockSpec((1,), lambda i: (0,)),
    ],
    out_specs=pl.BlockSpec((_BLK,), lambda i: (i,)),
    out_shape=jax.ShapeDtypeStruct((_BATCH,), jnp.float32),
)


def kernel(user_id, movie_id, user_emb, movie_emb, W1, b1, W2, b2, W3, b3, W4, b4):
    uid = user_id.astype(jnp.int32)
    mid = movie_id.astype(jnp.int32)
    gath = _sc_gather(uid, mid, user_emb.T, movie_emb.T)
    return _mlp_call(gath, W1, b1, W2, b2, W3, b3, W4, b4)


# speedup vs baseline: 2.5923x; 1.0920x over previous
import functools

import jax
import jax.numpy as jnp
from jax import lax
from jax.experimental import pallas as pl
from jax.experimental.pallas import tpu as pltpu
from jax.experimental.pallas import tpu_sc as plsc

_BATCH = 16384
_EMB = 32
_LANE = 128
_USERS = 1000000
_MOVIES = 100000
_NC = 2
_NS = 16
_NW = _NC * _NS
_BPW = _BATCH // _NW
_G = 4
_NGRP = _BPW // _G
_SROWS = 128

_UCMAX = ((_USERS - 1) // _LANE) * _LANE
_MCMAX = ((_MOVIES - 1) // _LANE) * _LANE

_mesh = plsc.VectorSubcoreMesh(core_axis_name="c", subcore_axis_name="s")


@functools.partial(
    pl.kernel,
    mesh=_mesh,
    compiler_params=pltpu.CompilerParams(needs_layout_passes=False),
    out_type=jax.ShapeDtypeStruct((_BATCH, _LANE), jnp.float32),
    scratch_types=[
        pltpu.VMEM((_BPW + 16,), jnp.int32),
        pltpu.VMEM((_BPW + 16,), jnp.int32),
        pltpu.VMEM((3, _G, _EMB, _LANE), jnp.float32),
        pltpu.VMEM((3, _G, _EMB, _LANE), jnp.float32),
        pltpu.VMEM((_SROWS, _LANE), jnp.float32),
        pltpu.SemaphoreType.DMA,
        pltpu.SemaphoreType.DMA,
        pltpu.SemaphoreType.DMA,
        pltpu.SemaphoreType.DMA,
        pltpu.SemaphoreType.DMA,
        pltpu.SemaphoreType.DMA,
    ],
)
def _sc_gather(uid_hbm, mid_hbm, ut_hbm, mt_hbm, out_hbm,
               uidx, midx, ublk, mblk, stage, su0, su1, su2, sm0, sm1, sm2):
    wid = lax.axis_index("s") * _NC + lax.axis_index("c")
    base = wid * _BPW
    pltpu.sync_copy(uid_hbm.at[pl.ds(base, _BPW)], uidx.at[pl.ds(0, _BPW)])
    pltpu.sync_copy(mid_hbm.at[pl.ds(base, _BPW)], midx.at[pl.ds(0, _BPW)])
    sems_u = (su0, su1, su2)
    sems_m = (sm0, sm1, sm2)
    iota = lax.iota(jnp.int32, 16)

    def blk_start(v, cmax):
        return jnp.minimum(v - (v & (_LANE - 1)), cmax)

    def fire(g, slot):
        uv = uidx[pl.ds(g * _G, 16)]
        mv = midx[pl.ds(g * _G, 16)]
        for j in range(_G):
            u = uv[j]
            cs = pl.multiple_of(blk_start(u, _UCMAX), _LANE)
            pltpu.async_copy(ut_hbm.at[:, pl.ds(cs, _LANE)],
                             ublk.at[slot, j], sems_u[slot])
            m = mv[j]
            ms = pl.multiple_of(blk_start(m, _MCMAX), _LANE)
            pltpu.async_copy(mt_hbm.at[:, pl.ds(ms, _LANE)],
                             mblk.at[slot, j], sems_m[slot])

    def drain(slot):
        for j in range(_G):
            pltpu.make_async_copy(ut_hbm.at[:, pl.ds(0, _LANE)],
                                  ublk.at[slot, j], sems_u[slot]).wait()
            pltpu.make_async_copy(mt_hbm.at[:, pl.ds(0, _LANE)],
                                  mblk.at[slot, j], sems_m[slot]).wait()

    def extract(blk_ref, col, row, colbase):
        cvec = jnp.full((16,), col, dtype=jnp.int32)
        rvec = jnp.full((16,), row, dtype=jnp.int32)
        lo = plsc.load_gather(blk_ref, [iota, cvec])
        hi = plsc.load_gather(blk_ref, [iota + 16, cvec])
        plsc.store_scatter(stage, [rvec, iota + colbase], lo)
        plsc.store_scatter(stage, [rvec, iota + (colbase + 16)], hi)

    def process(g, slot):
        uv = uidx[pl.ds(g * _G, 16)]
        mv = midx[pl.ds(g * _G, 16)]
        for j in range(_G):
            p = g * _G + j
            row = p & (_SROWS - 1)
            u = uv[j]
            ucol = u - blk_start(u, _UCMAX)
            extract(ublk.at[slot, j], ucol, row, 0)
            m = mv[j]
            mcol = m - blk_start(m, _MCMAX)
            extract(mblk.at[slot, j], mcol, row, 32)

    def flush(g):
        @pl.when((g & 31) == 31)
        def _():
            off = pl.multiple_of(base + (g // 32) * _SROWS, _SROWS)
            pltpu.sync_copy(stage, out_hbm.at[pl.ds(off, _SROWS)])

    fire(0, 0)
    fire(1, 1)
    fire(2, 2)

    def body(i, carry):
        g0 = 3 * i
        g1 = 3 * i + 1
        g2 = 3 * i + 2

        drain(0)
        process(g0, 0)
        flush(g0)

        @pl.when(g0 + 3 < _NGRP)
        def _():
            fire(g0 + 3, 0)

        drain(1)
        process(g1, 1)
        flush(g1)

        @pl.when(g1 + 3 < _NGRP)
        def _():
            fire(g1 + 3, 1)

        @pl.when(g2 < _NGRP)
        def _():
            drain(2)
            process(g2, 2)
            flush(g2)

        @pl.when(g2 + 3 < _NGRP)
        def _():
            fire(g2 + 3, 2)

        return carry

    lax.fori_loop(0, (_NGRP + 2) // 3, body, 0)


_BLK = 2048


def _mlp_body(gath, w1, b1, w2, b2, w3, b3, w4, b4, out):
    uf = gath[:, 0:_EMB]
    mf = gath[:, _EMB:2 * _EMB]
    w1u = w1[0:_EMB, :]
    w1m = w1[_EMB:2 * _EMB, :]
    x = jnp.dot(uf, w1u, preferred_element_type=jnp.float32)
    x = x + jnp.dot(mf, w1m, preferred_element_type=jnp.float32)
    x = jnp.maximum(x + b1[...][None, :], 0.0)
    x = jnp.maximum(
        jnp.dot(x, w2[...], preferred_element_type=jnp.float32)
        + b2[...][None, :], 0.0)
    x = jnp.maximum(
        jnp.dot(x, w3[...], preferred_element_type=jnp.float32)
        + b3[...][None, :], 0.0)
    y = jnp.dot(x, w4[...], preferred_element_type=jnp.float32)
    out[...] = y[:, 0] + b4[...]


_mlp_call = pl.pallas_call(
    _mlp_body,
    grid=(_BATCH // _BLK,),
    in_specs=[
        pl.BlockSpec((_BLK, _LANE), lambda i: (i, 0)),
        pl.BlockSpec((2 * _EMB, 32), lambda i: (0, 0)),
        pl.BlockSpec((32,), lambda i: (0,)),
        pl.BlockSpec((32, 16), lambda i: (0, 0)),
        pl.BlockSpec((16,), lambda i: (0,)),
        pl.BlockSpec((16, 8), lambda i: (0, 0)),
        pl.BlockSpec((8,), lambda i: (0,)),
        pl.BlockSpec((8, 1), lambda i: (0, 0)),
        pl.BlockSpec((1,), lambda i: (0,)),
    ],
    out_specs=pl.BlockSpec((_BLK,), lambda i: (i,)),
    out_shape=jax.ShapeDtypeStruct((_BATCH,), jnp.float32),
)


def kernel(user_id, movie_id, user_emb, movie_emb, W1, b1, W2, b2, W3, b3, W4, b4):
    uid = user_id.astype(jnp.int32)
    mid = movie_id.astype(jnp.int32)
    gath = _sc_gather(uid, mid, user_emb.T, movie_emb.T)
    return _mlp_call(gath, W1, b1, W2, b2, W3, b3, W4, b4)
